# baseline (speedup 1.0000x reference)
import functools

import jax
import jax.numpy as jnp
from jax import lax
from jax.experimental import pallas as pl
from jax.experimental.pallas import tpu as pltpu
from jax.experimental.pallas import tpu_sc as plsc

VOCAB = 1000000
EMBED = 32
BATCH = 4096
SEQ = 200

NC = 2
NS = 16
NW = NC * NS
LT = SEQ // 8
HALF = 4
N_STEPS = SEQ // HALF
ROWS = HALF * 128


@functools.partial(
    pl.kernel,
    out_type=jax.ShapeDtypeStruct((SEQ, EMBED // 8, NW, 8, 128),
                                  jnp.float32),
    mesh=plsc.VectorSubcoreMesh(
        core_axis_name="c", subcore_axis_name="s",
        num_cores=NC, num_subcores=NS),
    scratch_types=[
        pltpu.VMEM((HALF, 128), jnp.int32),
        pltpu.VMEM((HALF, 128), jnp.int32),
        pltpu.VMEM((ROWS, EMBED), jnp.float32),
        pltpu.VMEM((ROWS, EMBED), jnp.float32),
        pltpu.VMEM((HALF, EMBED // 8, 8, 128), jnp.float32),
        pltpu.VMEM((HALF, EMBED // 8, 8, 128), jnp.float32),
        pltpu.SemaphoreType.DMA,
        pltpu.SemaphoreType.DMA,
        pltpu.SemaphoreType.DMA,
        pltpu.SemaphoreType.DMA,
        pltpu.SemaphoreType.DMA,
        pltpu.SemaphoreType.DMA,
    ],
    compiler_params=pltpu.CompilerParams(use_tc_tiling_on_sc=False,
                                         needs_layout_passes=False),
)
def _gather_kernel(table_hbm, idx_hbm, out_hbm,
                   i0, i1, r0, r1, t0, t1, si0, si1, sg0, sg1, so0, so1):
    idx_v = [i0, i1]
    rows_v = [r0, r1]
    rt_v = [t0, t1]
    isem = [si0, si1]
    gsem = [sg0, sg1]
    osem = [so0, so1]

    wid = lax.axis_index("s") * NC + lax.axis_index("c")
    iota16 = lax.iota(jnp.int32, 16)

    def idx_src(t):
        return idx_hbm.at[t // 2, wid, pl.ds((t % 2) * HALF, HALF)]

    def out_dst(t):
        return out_hbm.at[pl.ds(t * HALF, HALF), :, wid]

    pltpu.async_copy(idx_src(0), idx_v[0], isem[0])

    @pl.loop(0, N_STEPS // 2)
    def _pair(jj):
        for p in range(2):
            t = jj * 2 + p
            pltpu.make_async_copy(idx_src(t), idx_v[p], isem[p]).wait()

            @pl.when(jj > 0)
            def _():
                pltpu.make_async_copy(rt_v[p], out_dst(t - 2),
                                      osem[p]).wait()

            for i in range(HALF):
                pltpu.async_copy(
                    table_hbm.at[idx_v[p].at[i]],
                    rows_v[p].at[pl.ds(i * 128, 128)],
                    gsem[p])
            if p == 0:
                pltpu.async_copy(idx_src(t + 1), idx_v[1], isem[1])
            else:
                @pl.when(jj < N_STEPS // 2 - 1)
                def _():
                    pltpu.async_copy(idx_src(t + 1), idx_v[0], isem[0])
            for i in range(HALF):
                pltpu.make_async_copy(
                    table_hbm.at[idx_v[p].at[0]],
                    rows_v[p].at[pl.ds(0, 128)],
                    gsem[p]).wait()

            @pl.loop(0, EMBED)
            def _e(e):
                eb = e // 8
                es = e % 8
                ecol = jnp.full((16,), e, jnp.int32)
                for ls in range(HALF):
                    for jb in range(8):
                        ridx = ls * 128 + jb * 16 + iota16
                        vals = plsc.load_gather(rows_v[p], [ridx, ecol])
                        rt_v[p][ls, eb, es, pl.ds(jb * 16, 16)] = vals

            pltpu.async_copy(rt_v[p], out_dst(t), osem[p])

    for t in range(N_STEPS - 2, N_STEPS):
        pltpu.make_async_copy(rt_v[t % 2], out_dst(t), osem[t % 2]).wait()


def kernel(indices, table):
    idx_phys = (indices.T.reshape(LT, 8, NW, 128)
                .transpose(0, 2, 1, 3))
    out_phys = _gather_kernel(table, idx_phys)
    return (out_phys.transpose(2, 4, 0, 1, 3)
            .reshape(BATCH, SEQ, EMBED))


# speedup vs baseline: 1.1917x; 1.1917x over previous
import functools

import jax
import jax.numpy as jnp
from jax import lax
from jax.experimental import pallas as pl
from jax.experimental.pallas import tpu as pltpu
from jax.experimental.pallas import tpu_sc as plsc

VOCAB = 1000000
EMBED = 32
BATCH = 4096
SEQ = 200

NC = 2
NS = 16
NW = NC * NS
LT = SEQ // 8
HALF = 4
N_STEPS = SEQ // HALF
ROWS = HALF * 128


@functools.partial(
    pl.kernel,
    out_type=jax.ShapeDtypeStruct((BATCH, SEQ, EMBED), jnp.float32),
    mesh=plsc.VectorSubcoreMesh(
        core_axis_name="c", subcore_axis_name="s",
        num_cores=NC, num_subcores=NS),
    scratch_types=[
        pltpu.VMEM((HALF, 128), jnp.int32),
        pltpu.VMEM((HALF, 128), jnp.int32),
        pltpu.VMEM((ROWS, EMBED), jnp.float32),
        pltpu.VMEM((ROWS, EMBED), jnp.float32),
        pltpu.SemaphoreType.DMA,
        pltpu.SemaphoreType.DMA,
        pltpu.SemaphoreType.DMA,
        pltpu.SemaphoreType.DMA,
        pltpu.SemaphoreType.DMA,
        pltpu.SemaphoreType.DMA,
    ],
    compiler_params=pltpu.CompilerParams(use_tc_tiling_on_sc=False),
)
def _gather_kernel(table_hbm, idx_hbm, out_hbm,
                   i0, i1, r0, r1, si0, si1, sg0, sg1, so0, so1):
    idx_v = [i0, i1]
    rows_v = [r0, r1]
    isem = [si0, si1]
    gsem = [sg0, sg1]
    osem = [so0, so1]

    wid = lax.axis_index("s") * NC + lax.axis_index("c")
    b0 = wid * 128

    def idx_src(t):
        return idx_hbm.at[t // 2, wid, pl.ds((t % 2) * HALF, HALF)]

    def wb(t, p, sem):
        for i in range(HALF):
            pltpu.async_copy(
                rows_v[p].at[pl.ds(i * 128, 128)],
                out_hbm.at[pl.ds(b0, 128), t * HALF + i],
                sem)

    def wb_wait(t, p, sem):
        for i in range(HALF):
            pltpu.make_async_copy(
                rows_v[p].at[pl.ds(0, 128)],
                out_hbm.at[pl.ds(b0, 128), 0],
                sem).wait()

    pltpu.async_copy(idx_src(0), idx_v[0], isem[0])

    @pl.loop(0, N_STEPS // 2)
    def _pair(jj):
        for p in range(2):
            t = jj * 2 + p
            pltpu.make_async_copy(idx_src(t), idx_v[p], isem[p]).wait()

            @pl.when(jj > 0)
            def _():
                wb_wait(t - 2, p, osem[p])

            for i in range(HALF):
                pltpu.async_copy(
                    table_hbm.at[idx_v[p].at[i]],
                    rows_v[p].at[pl.ds(i * 128, 128)],
                    gsem[p])
            if p == 0:
                pltpu.async_copy(idx_src(t + 1), idx_v[1], isem[1])
            else:
                @pl.when(jj < N_STEPS // 2 - 1)
                def _():
                    pltpu.async_copy(idx_src(t + 1), idx_v[0], isem[0])
            for i in range(HALF):
                pltpu.make_async_copy(
                    table_hbm.at[idx_v[p].at[0]],
                    rows_v[p].at[pl.ds(0, 128)],
                    gsem[p]).wait()

            wb(t, p, osem[p])

    for t in range(N_STEPS - 2, N_STEPS):
        wb_wait(t, t % 2, osem[t % 2])


def kernel(indices, table):
    idx_phys = (indices.T.reshape(LT, 8, NW, 128)
                .transpose(0, 2, 1, 3))
    return _gather_kernel(table, idx_phys)
